# baseline (speedup 1.0000x reference)
import functools

import jax
import jax.numpy as jnp
from jax import lax
from jax.experimental import pallas as pl
from jax.experimental.pallas import tpu as pltpu
from jax.experimental.pallas import tpu_sc as plsc

SCALE = 0.5
CHUNK = 128
NBUF = 5


def _fuse_table(table3, At3, Bm, sub):
    n_maj, n_min, H = table3.shape
    R = At3.shape[0]
    assert n_maj % sub == 0

    def body(t_ref, at_ref, b_ref, o_ref):
        for k in range(sub):
            delta = jax.lax.dot_general(
                at_ref[:, k, :], b_ref[...], (((0,), (0,)), ((), ())),
                preferred_element_type=jnp.float32)
            o_ref[k] = t_ref[k] + delta * SCALE

    return pl.pallas_call(
        body,
        grid=(n_maj // sub,),
        in_specs=[
            pl.BlockSpec((sub, n_min, H), lambda i: (i, 0, 0)),
            pl.BlockSpec((R, sub, n_min), lambda i: (0, i, 0)),
            pl.BlockSpec((R, H), lambda i: (0, 0)),
        ],
        out_specs=pl.BlockSpec((sub, n_min, H), lambda i: (i, 0, 0)),
        out_shape=jax.ShapeDtypeStruct((n_maj, n_min, H), jnp.float32),
    )(table3, At3, Bm)


@functools.lru_cache(maxsize=None)
def _make_gather(V, H, BT):
    info = plsc.get_sparse_core_info()
    NC, NS = info.num_cores, info.num_subcores
    NW = NC * NS
    assert BT % (NW * CHUNK) == 0
    per_tile = BT // NW
    n_chunks = per_tile // CHUNK
    assert n_chunks % NBUF == 0
    mesh = plsc.VectorSubcoreMesh(core_axis_name="c", subcore_axis_name="s")

    @functools.partial(
        pl.kernel,
        mesh=mesh,
        out_type=jax.ShapeDtypeStruct((BT, H), jnp.float32),
        scratch_types=[
            pltpu.VMEM((per_tile,), jnp.int32),
            pltpu.VMEM((NBUF, CHUNK, H), jnp.float32),
        ]
        + [pltpu.SemaphoreType.DMA] * (2 * NBUF),
    )
    def gather_k(fused_hbm, ids_hbm, out_hbm, idx_v, rows_v, *sems):
        gsem = sems[:NBUF]
        ssem = sems[NBUF:]
        c = lax.axis_index("c")
        s = lax.axis_index("s")
        wid = s * NC + c
        base = wid * per_tile
        pltpu.sync_copy(ids_hbm.at[pl.ds(base, per_tile)], idx_v)

        def gather_copy(j, b):
            return pltpu.make_async_copy(
                fused_hbm.at[idx_v.at[pl.ds(j * CHUNK, CHUNK)]],
                rows_v.at[b], gsem[b])

        def store_copy(j, b):
            return pltpu.make_async_copy(
                rows_v.at[b],
                out_hbm.at[pl.ds(base + j * CHUNK, CHUNK)], ssem[b])

        for b in range(NBUF):
            gather_copy(b, b).start()

        def outer(j0, carry):
            for b in range(NBUF):
                j = j0 + b
                gather_copy(j, b).wait()
                store_copy(j, b).start()

                @pl.when(j + NBUF < n_chunks)
                def _():
                    store_copy(j, b).wait()
                    gather_copy(j + NBUF, b).start()

            return carry

        lax.fori_loop(0, n_chunks // NBUF, lambda i, c: outer(i * NBUF, c), 0)

        for b in range(NBUF):
            store_copy(n_chunks - NBUF + b, b).wait()

    return gather_k


def kernel(ids, table, A, Bm):
    V, H = table.shape
    B, S = ids.shape
    n_maj, n_min = 160, V // 160
    fused3 = _fuse_table(
        table.reshape(n_maj, n_min, H),
        A.T.reshape(A.shape[1], n_maj, n_min), Bm, sub=16)
    fused = fused3.reshape(V, H)
    gather = _make_gather(V, H, B * S)
    out_t = gather(fused, ids.T.reshape(-1))
    return out_t.reshape(S, B, H).transpose(1, 0, 2)


# speedup vs baseline: 1.5089x; 1.5089x over previous
import functools

import jax
import jax.numpy as jnp
from jax import lax
from jax.experimental import pallas as pl
from jax.experimental.pallas import tpu as pltpu
from jax.experimental.pallas import tpu_sc as plsc

SCALE = 0.5
CHUNK = 128
NBUF = 5


def _fuse_table(table, At3, Bm):
    V, H = table.shape
    R, n_maj, n_min = At3.shape
    blk = 10000
    assert V % blk == 0 and blk % n_min == 0
    sub = blk // n_min
    assert sub % 8 == 0

    def body(t_ref, at_ref, b_ref, o_ref):
        at = at_ref[...].reshape(R, blk)
        delta = jax.lax.dot_general(
            at, b_ref[...], (((0,), (0,)), ((), ())),
            preferred_element_type=jnp.float32)
        o_ref[...] = t_ref[...] + delta * SCALE

    return pl.pallas_call(
        body,
        grid=(V // blk,),
        in_specs=[
            pl.BlockSpec((blk, H), lambda i: (i, 0)),
            pl.BlockSpec((R, sub, n_min), lambda i: (0, i, 0)),
            pl.BlockSpec((R, H), lambda i: (0, 0)),
        ],
        out_specs=pl.BlockSpec((blk, H), lambda i: (i, 0)),
        out_shape=jax.ShapeDtypeStruct((V, H), jnp.float32),
    )(table, At3, Bm)


@functools.lru_cache(maxsize=None)
def _make_gather(V, H, BT):
    info = plsc.get_sparse_core_info()
    NC, NS = info.num_cores, info.num_subcores
    NW = NC * NS
    assert BT % (NW * CHUNK) == 0
    per_tile = BT // NW
    n_chunks = per_tile // CHUNK
    assert n_chunks % NBUF == 0
    mesh = plsc.VectorSubcoreMesh(core_axis_name="c", subcore_axis_name="s")

    @functools.partial(
        pl.kernel,
        mesh=mesh,
        out_type=jax.ShapeDtypeStruct((BT, H), jnp.float32),
        scratch_types=[
            pltpu.VMEM((per_tile,), jnp.int32),
            pltpu.VMEM((NBUF, CHUNK, H), jnp.float32),
        ]
        + [pltpu.SemaphoreType.DMA] * (2 * NBUF),
    )
    def gather_k(fused_hbm, ids_hbm, out_hbm, idx_v, rows_v, *sems):
        gsem = sems[:NBUF]
        ssem = sems[NBUF:]
        c = lax.axis_index("c")
        s = lax.axis_index("s")
        wid = s * NC + c
        base = wid * per_tile
        pltpu.sync_copy(ids_hbm.at[pl.ds(base, per_tile)], idx_v)

        def gather_copy(j, b):
            return pltpu.make_async_copy(
                fused_hbm.at[idx_v.at[pl.ds(j * CHUNK, CHUNK)]],
                rows_v.at[b], gsem[b])

        def store_copy(j, b):
            return pltpu.make_async_copy(
                rows_v.at[b],
                out_hbm.at[pl.ds(base + j * CHUNK, CHUNK)], ssem[b])

        for b in range(NBUF):
            gather_copy(b, b).start()

        def outer(j0, carry):
            for b in range(NBUF):
                j = j0 + b
                gather_copy(j, b).wait()
                store_copy(j, b).start()

                @pl.when(j + NBUF < n_chunks)
                def _():
                    store_copy(j, b).wait()
                    gather_copy(j + NBUF, b).start()

            return carry

        lax.fori_loop(0, n_chunks // NBUF, lambda i, c: outer(i * NBUF, c), 0)

        for b in range(NBUF):
            store_copy(n_chunks - NBUF + b, b).wait()

    return gather_k


def kernel(ids, table, A, Bm):
    V, H = table.shape
    B, S = ids.shape
    n_maj, n_min = 80, V // 80
    fused = _fuse_table(
        table, A.T.reshape(A.shape[1], n_maj, n_min), Bm)
    gather = _make_gather(V, H, B * S)
    out_t = gather(fused, ids.T.reshape(-1))
    return out_t.reshape(S, B, H).transpose(1, 0, 2)


# speedup vs baseline: 1.5175x; 1.0057x over previous
import functools

import jax
import jax.numpy as jnp
from jax import lax
from jax.experimental import pallas as pl
from jax.experimental.pallas import tpu as pltpu
from jax.experimental.pallas import tpu_sc as plsc

SCALE = 0.5
CHUNK = 80
NBUF = 8


def _fuse_table(table, At3, Bm):
    V, H = table.shape
    R, n_maj, n_min = At3.shape
    blk = 10000
    assert V % blk == 0 and blk % n_min == 0
    sub = blk // n_min
    assert sub % 8 == 0

    def body(t_ref, at_ref, b_ref, o_ref):
        at = at_ref[...].reshape(R, blk)
        delta = jax.lax.dot_general(
            at, b_ref[...], (((0,), (0,)), ((), ())),
            preferred_element_type=jnp.float32)
        o_ref[...] = t_ref[...] + delta * SCALE

    return pl.pallas_call(
        body,
        grid=(V // blk,),
        in_specs=[
            pl.BlockSpec((blk, H), lambda i: (i, 0)),
            pl.BlockSpec((R, sub, n_min), lambda i: (0, i, 0)),
            pl.BlockSpec((R, H), lambda i: (0, 0)),
        ],
        out_specs=pl.BlockSpec((blk, H), lambda i: (i, 0)),
        out_shape=jax.ShapeDtypeStruct((V, H), jnp.float32),
    )(table, At3, Bm)


@functools.lru_cache(maxsize=None)
def _make_gather(V, H, BT):
    info = plsc.get_sparse_core_info()
    NC, NS = info.num_cores, info.num_subcores
    NW = NC * NS
    assert BT % (NW * CHUNK) == 0
    per_tile = BT // NW
    n_chunks = per_tile // CHUNK
    assert n_chunks % NBUF == 0
    mesh = plsc.VectorSubcoreMesh(core_axis_name="c", subcore_axis_name="s")

    @functools.partial(
        pl.kernel,
        mesh=mesh,
        out_type=jax.ShapeDtypeStruct((BT, H), jnp.float32),
        scratch_types=[
            pltpu.VMEM((per_tile,), jnp.int32),
            pltpu.VMEM((NBUF, CHUNK, H), jnp.float32),
        ]
        + [pltpu.SemaphoreType.DMA] * (2 * NBUF),
    )
    def gather_k(fused_hbm, ids_hbm, out_hbm, idx_v, rows_v, *sems):
        gsem = sems[:NBUF]
        ssem = sems[NBUF:]
        c = lax.axis_index("c")
        s = lax.axis_index("s")
        wid = s * NC + c
        base = wid * per_tile
        pltpu.sync_copy(ids_hbm.at[pl.ds(base, per_tile)], idx_v)

        def gather_copy(j, b):
            return pltpu.make_async_copy(
                fused_hbm.at[idx_v.at[pl.ds(j * CHUNK, CHUNK)]],
                rows_v.at[b], gsem[b])

        def store_copy(j, b):
            return pltpu.make_async_copy(
                rows_v.at[b],
                out_hbm.at[pl.ds(base + j * CHUNK, CHUNK)], ssem[b])

        for b in range(NBUF):
            gather_copy(b, b).start()

        def outer(j0, carry):
            for b in range(NBUF):
                j = j0 + b
                gather_copy(j, b).wait()
                store_copy(j, b).start()

                @pl.when(j + NBUF < n_chunks)
                def _():
                    store_copy(j, b).wait()
                    gather_copy(j + NBUF, b).start()

            return carry

        lax.fori_loop(0, n_chunks // NBUF, lambda i, c: outer(i * NBUF, c), 0)

        for b in range(NBUF):
            store_copy(n_chunks - NBUF + b, b).wait()

    return gather_k


def kernel(ids, table, A, Bm):
    V, H = table.shape
    B, S = ids.shape
    n_maj, n_min = 80, V // 80
    fused = _fuse_table(
        table, A.T.reshape(A.shape[1], n_maj, n_min), Bm)
    gather = _make_gather(V, H, B * S)
    out_t = gather(fused, ids.T.reshape(-1))
    return out_t.reshape(S, B, H).transpose(1, 0, 2)
